# baseline (speedup 1.0000x reference)
import functools

import jax
import jax.numpy as jnp
from jax import lax
from jax.experimental import pallas as pl
from jax.experimental.pallas import tpu as pltpu
from jax.experimental.pallas import tpu_sc as plsc

VOCAB = 1000000
DIM = 64
BATCH = 16384

_NUM_WORKERS = 32
_RPW = BATCH // _NUM_WORKERS
_RING = 8
_L = 16


@functools.partial(
    pl.kernel,
    mesh=plsc.VectorSubcoreMesh(core_axis_name="c", subcore_axis_name="s"),
    out_type=jax.ShapeDtypeStruct((DIM, BATCH), jnp.float32),
    scratch_types=[
        pltpu.VMEM((_RPW,), jnp.int32),
        pltpu.VMEM((DIM, 128), jnp.float32),
        pltpu.VMEM((DIM, _RPW), jnp.float32),
        pltpu.SemaphoreType.DMA,
    ],
    compiler_params=pltpu.CompilerParams(needs_layout_passes=False),
)
def _gather_kernel(idx_hbm, table_hbm, out_hbm, idx_v, panel_v, cols_v, sem):
    wid = lax.axis_index("s") * 2 + lax.axis_index("c")
    base = wid * _RPW
    pltpu.sync_copy(idx_hbm.at[pl.ds(base, _RPW)], idx_v)

    lane = lax.iota(jnp.int32, _L)

    def chunk(ci, _):
        vec = idx_v[pl.ds(ci * _L, _L)]
        panel = lax.shift_right_logical(vec, 7)
        col = lax.bitwise_and(vec, 127)
        for l in range(_L):
            c0 = pl.multiple_of(panel[l] * 128, 128)
            copy = pltpu.make_async_copy(
                table_hbm.at[:, pl.ds(c0, 128)], panel_v, sem
            )
            copy.start()
            copy.wait()
            j = ci * _L + l
            vcol = jnp.full((_L,), col[l], dtype=jnp.int32)
            jcol = jnp.full((_L,), j, dtype=jnp.int32)
            for k in range(DIM // _L):
                d = lane + k * _L
                v = plsc.load_gather(panel_v, [d, vcol])
                plsc.store_scatter(cols_v, [d, jcol], v)
        return 0

    lax.fori_loop(0, _RPW // _L, chunk, 0)
    pltpu.sync_copy(cols_v, out_hbm.at[:, pl.ds(base, _RPW)])


def kernel(indices, table):
    out_t = _gather_kernel(indices, table.T)
    return out_t.T


# speedup vs baseline: 2.0677x; 2.0677x over previous
import functools

import jax
import jax.numpy as jnp
from jax import lax
from jax.experimental import pallas as pl
from jax.experimental.pallas import tpu as pltpu
from jax.experimental.pallas import tpu_sc as plsc

VOCAB = 1000000
DIM = 64
BATCH = 16384

_NUM_WORKERS = 32
_RPW = BATCH // _NUM_WORKERS
_L = 16
_RING = 8


@functools.partial(
    pl.kernel,
    mesh=plsc.VectorSubcoreMesh(core_axis_name="c", subcore_axis_name="s"),
    out_type=jax.ShapeDtypeStruct((DIM, BATCH), jnp.float32),
    scratch_types=[
        pltpu.VMEM((_RPW,), jnp.int32),
        pltpu.VMEM((DIM, _RING * 128), jnp.float32),
        pltpu.VMEM((DIM, 128), jnp.float32),
        pltpu.SemaphoreType.DMA,
    ],
    compiler_params=pltpu.CompilerParams(needs_layout_passes=False),
)
def _gather_kernel(idx_hbm, table_hbm, out_hbm, idx_v, ring_v, cols_v, sem):
    wid = lax.axis_index("s") * 2 + lax.axis_index("c")
    base = wid * _RPW
    pltpu.sync_copy(idx_hbm.at[pl.ds(base, _RPW)], idx_v)

    lane = lax.iota(jnp.int32, _L)

    def chunk(ci, _):
        vec = idx_v[pl.ds(ci * _L, _L)]
        panel = lax.shift_right_logical(vec, 7)
        col = lax.bitwise_and(vec, 127)
        for half in range(_L // _RING):
            for s in range(_RING):
                l = half * _RING + s
                c0 = pl.multiple_of(panel[l] * 128, 128)
                pltpu.make_async_copy(
                    table_hbm.at[:, pl.ds(c0, 128)],
                    ring_v.at[:, pl.ds(s * 128, 128)],
                    sem,
                ).start()
            pltpu.make_async_copy(
                table_hbm.at[:, pl.ds(0, _RING * 128)], ring_v, sem
            ).wait()
            for s in range(_RING):
                l = half * _RING + s
                vcol = jnp.full((_L,), s * 128, dtype=jnp.int32) + col[l]
                jcol = jnp.full((_L,), (ci * _L + l) % 128, dtype=jnp.int32)
                for k in range(DIM // _L):
                    d = lane + k * _L
                    v = plsc.load_gather(ring_v, [d, vcol])
                    plsc.store_scatter(cols_v, [d, jcol], v)
        @pl.when((ci & 7) == 7)
        def _():
            j0 = pl.multiple_of((ci - 7) * _L, 128)
            pltpu.sync_copy(cols_v, out_hbm.at[:, pl.ds(base + j0, 128)])

        return 0

    lax.fori_loop(0, _RPW // _L, chunk, 0)


def kernel(indices, table):
    out_t = _gather_kernel(indices, table.T)
    return out_t.T


# speedup vs baseline: 2.8465x; 1.3766x over previous
import functools

import jax
import jax.numpy as jnp
from jax import lax
from jax.experimental import pallas as pl
from jax.experimental.pallas import tpu as pltpu
from jax.experimental.pallas import tpu_sc as plsc

VOCAB = 1000000
DIM = 64
BATCH = 16384

_NUM_WORKERS = 32
_RPW = BATCH // _NUM_WORKERS
_L = 16
_G = 4
_NG = _RPW // _G


@functools.partial(
    pl.kernel,
    mesh=plsc.VectorSubcoreMesh(core_axis_name="c", subcore_axis_name="s"),
    out_type=jax.ShapeDtypeStruct((DIM, BATCH), jnp.float32),
    scratch_types=[
        pltpu.VMEM((_RPW + 16,), jnp.int32),
        pltpu.VMEM((DIM, 3 * _G * 128), jnp.float32),
        pltpu.VMEM((DIM, 128), jnp.float32),
        pltpu.SemaphoreType.DMA,
    ],
    compiler_params=pltpu.CompilerParams(needs_layout_passes=False),
)
def _gather_kernel(idx_hbm, table_hbm, out_hbm, idx_v, ring_v, cols_v, sem):
    wid = lax.axis_index("s") * 2 + lax.axis_index("c")
    base = wid * _RPW
    pltpu.sync_copy(idx_hbm.at[pl.ds(base, _RPW)], idx_v.at[pl.ds(0, _RPW)])

    lane = lax.iota(jnp.int32, _L)

    def fire(g, par):
        vec = idx_v[pl.ds(g * _G, _L)]
        panel = lax.shift_right_logical(vec, 7)
        for r in range(_G):
            c0 = pl.multiple_of(panel[r] * 128, 128)
            pltpu.make_async_copy(
                table_hbm.at[:, pl.ds(c0, 128)],
                ring_v.at[:, pl.ds((par * _G + r) * 128, 128)],
                sem,
            ).start()

    def drain(par):
        pltpu.make_async_copy(
            table_hbm.at[:, pl.ds(0, _G * 128)],
            ring_v.at[:, pl.ds(par * _G * 128, _G * 128)],
            sem,
        ).wait()

    def extract(g, par):
        vec = idx_v[pl.ds(g * _G, _L)]
        col = lax.bitwise_and(vec, 127)
        for r in range(_G):
            vcol = jnp.full((_L,), (par * _G + r) * 128, jnp.int32) + col[r]
            jcol = jnp.full((_L,), 0, jnp.int32) + ((g * _G + r) & 127)
            for k in range(DIM // _L):
                d = lane + k * _L
                v = plsc.load_gather(ring_v, [d, vcol])
                plsc.store_scatter(cols_v, [d, jcol], v)
        @pl.when((g & 31) == 31)
        def _():
            j0 = pl.multiple_of(g * _G - 124, 128)
            pltpu.sync_copy(cols_v, out_hbm.at[:, pl.ds(base + j0, 128)])

    fire(0, 0)
    fire(1, 1)

    def step(si, _):
        for sub in range(3):
            g = si * 3 + sub
            drain(sub)
            fire(g + 2, (sub + 2) % 3)
            extract(g, sub)
        return 0

    lax.fori_loop(0, (_NG - 2) // 3, step, 0)
    drain(0)
    extract(_NG - 2, 0)
    drain(1)
    extract(_NG - 1, 1)


def kernel(indices, table):
    out_t = _gather_kernel(indices, table.T)
    return out_t.T
